# Initial kernel scaffold; baseline (speedup 1.0000x reference)
import jax
import jax.numpy as jnp
from jax.experimental import pallas as pl


def kernel(x, W1, b1, W2, b2, codebook):
    raise NotImplementedError("write your pallas kernel here")


# speedup vs baseline: 2.0545x; 2.0545x over previous
import jax
import jax.numpy as jnp
from jax.experimental import pallas as pl
from jax.experimental.pallas import tpu as pltpu

B = 4
C_IN = 256
T = 32768
C1 = 128
C2 = 64
K_EMB = 1024
D_EMB = 64
BETA = 0.25
T2 = 8190
N_ROWS = B * T2
W2B = 2048
NB = (T2 + W2B - 1) // W2B
W1B = 2 * W2B + 2
WMAIN = 4 * W2B
WXP = WMAIN + 128
TSUB = 512
NCH = W2B // TSUB
NLAST = T2 - (NB - 1) * W2B


def _issue_x_copy(x_hbm, xwin, sem_in, b, i, slot):
    main = pltpu.make_async_copy(
        x_hbm.at[b, :, pl.ds(WMAIN * i, WMAIN)],
        xwin.at[slot, :, pl.ds(0, WMAIN)], sem_in.at[slot])
    tail_off = 128 * jnp.minimum((WMAIN // 128) * i + WMAIN // 128, T // 128 - 1)
    tail = pltpu.make_async_copy(
        x_hbm.at[b, :, pl.ds(tail_off, 128)],
        xwin.at[slot, :, pl.ds(WMAIN, 128)], sem_in.at[slot])
    return main, tail


def _vq_kernel(x_hbm, w1t_ref, b1_ref, w2t_ref, b2_ref, cb_ref, cb2_ref,
               cs_ref, out_hbm, loss_ref, perp_ref,
               xwin, acol, bcol, h1s, qout, counts, msacc, sem_in, sem_out):
    b = pl.program_id(0)
    i = pl.program_id(1)
    step = b * NB + i
    par = jax.lax.rem(step, 2)
    j0 = i * W2B
    nvalid = jnp.where(i == NB - 1, NLAST, W2B)

    @pl.when(step == 0)
    def _init():
        counts[...] = jnp.zeros_like(counts)
        msacc[0] = 0.0
        m0, t0 = _issue_x_copy(x_hbm, xwin, sem_in, b, i, par)
        m0.start()
        t0.start()

    @pl.when(step < B * NB - 1)
    def _prefetch():
        nb_ = jnp.where(i == NB - 1, b + 1, b)
        ni = jnp.where(i == NB - 1, 0, i + 1)
        m1, t1 = _issue_x_copy(x_hbm, xwin, sem_in, nb_, ni, 1 - par)
        m1.start()
        t1.start()

    mc, tc = _issue_x_copy(x_hbm, xwin, sem_in, b, i, par)
    mc.wait()
    tc.wait()

    xt = jax.lax.transpose(xwin[par], (1, 0))
    acol[...] = xt[:, :128]
    bcol[...] = xt[:, 128:]

    acc1 = jnp.zeros((W1B, C1), jnp.float32)
    for k in range(4):
        xa = acol[pl.Slice(k, W1B, 2), :]
        xb = bcol[pl.Slice(k, W1B, 2), :]
        acc1 = acc1 + jax.lax.dot_general(
            xa, w1t_ref[pl.ds(k * 256, 128), :], (((1,), (0,)), ((), ())),
            preferred_element_type=jnp.float32)
        acc1 = acc1 + jax.lax.dot_general(
            xb, w1t_ref[pl.ds(k * 256 + 128, 128), :], (((1,), (0,)), ((), ())),
            preferred_element_type=jnp.float32)
    h1s[pl.ds(0, W1B), :] = jnp.tanh(acc1 + b1_ref[...])

    @pl.when(step >= 2)
    def _drain_prev():
        pj = jnp.maximum(i - 2, 0) * W2B
        pltpu.make_async_copy(
            qout.at[par], out_hbm.at[b, :, pl.ds(pj, W2B)], sem_out.at[par]).wait()

    for c in range(NCH):
        acc2 = jnp.zeros((TSUB, C2), jnp.float32)
        for m in range(4):
            hm = h1s[pl.Slice(2 * c * TSUB + m, TSUB, 2), :]
            acc2 = acc2 + jax.lax.dot_general(
                hm, w2t_ref[pl.ds(m * 128, 128), :], (((1,), (0,)), ((), ())),
                preferred_element_type=jnp.float32)
        h2c = jnp.tanh(acc2 + b2_ref[...])

        rn = jnp.sum(h2c * h2c, axis=1, keepdims=True)
        mm2 = jax.lax.dot_general(
            h2c, cb2_ref[...], (((1,), (1,)), ((), ())),
            preferred_element_type=jnp.float32)
        dist = (rn + cs_ref[...]) - mm2

        minv = jnp.min(dist, axis=1, keepdims=True)
        lane = jax.lax.broadcasted_iota(jnp.int32, (TSUB, K_EMB), 1)
        idx = jnp.min(jnp.where(dist == minv, lane, K_EMB), axis=1, keepdims=True)
        onehot = (lane == idx).astype(jnp.float32)

        row = jax.lax.broadcasted_iota(jnp.int32, (TSUB, 1), 0)
        vmask = row < (nvalid - c * TSUB)
        counts[...] = counts[...] + jnp.sum(
            jnp.where(vmask, onehot, 0.0), axis=0, keepdims=True)

        quant = jax.lax.dot_general(
            onehot, cb_ref[...], (((1,), (0,)), ((), ())),
            preferred_element_type=jnp.float32)
        d = h2c - quant
        msacc[0] = msacc[0] + jnp.sum(jnp.where(vmask, d * d, 0.0))

        qfinal = h2c + (quant - h2c)
        qout[par, :, pl.ds(c * TSUB, TSUB)] = jax.lax.transpose(qfinal, (1, 0))

    ocp = pltpu.make_async_copy(
        qout.at[par], out_hbm.at[b, :, pl.ds(j0, W2B)], sem_out.at[par])
    ocp.start()

    @pl.when(step == B * NB - 1)
    def _fin():
        ocp.wait()
        pltpu.make_async_copy(
            qout.at[1 - par], out_hbm.at[b, :, pl.ds(j0 - W2B, W2B)],
            sem_out.at[1 - par]).wait()
        e = msacc[0] / jnp.float32(N_ROWS * D_EMB)
        loss_ref[0] = e + jnp.float32(BETA) * e
        p = counts[...] / jnp.float32(N_ROWS)
        ent = jnp.sum(p * jnp.log(p + 1e-10))
        perp_ref[0] = jnp.exp(-ent)


@jax.jit
def kernel(x, W1, b1, W2, b2, codebook):
    w1t = jnp.transpose(W1, (2, 1, 0)).reshape(4 * C_IN, C1)
    w2t = jnp.transpose(W2, (2, 1, 0)).reshape(4 * C1, C2)
    b1r = b1.reshape(1, C1)
    b2r = b2.reshape(1, C2)
    cb2 = codebook * 2.0
    cs = jnp.sum(codebook, axis=1).reshape(1, K_EMB)

    full = lambda shape: pl.BlockSpec(shape, lambda b, i: tuple(0 for _ in shape))
    out, loss, perp = pl.pallas_call(
        _vq_kernel,
        grid=(B, NB),
        in_specs=[
            pl.BlockSpec(memory_space=pl.ANY),
            full((4 * C_IN, C1)),
            full((1, C1)),
            full((4 * C1, C2)),
            full((1, C2)),
            full((K_EMB, D_EMB)),
            full((K_EMB, D_EMB)),
            full((1, K_EMB)),
        ],
        out_specs=[
            pl.BlockSpec(memory_space=pl.ANY),
            pl.BlockSpec(memory_space=pltpu.SMEM),
            pl.BlockSpec(memory_space=pltpu.SMEM),
        ],
        out_shape=[
            jax.ShapeDtypeStruct((B, C2, NB * W2B), jnp.float32),
            jax.ShapeDtypeStruct((1,), jnp.float32),
            jax.ShapeDtypeStruct((1,), jnp.float32),
        ],
        scratch_shapes=[
            pltpu.VMEM((2, C_IN, WXP), jnp.float32),
            pltpu.VMEM((WXP, 128), jnp.float32),
            pltpu.VMEM((WXP, 128), jnp.float32),
            pltpu.VMEM((W1B + 6, 128), jnp.float32),
            pltpu.VMEM((2, C2, W2B), jnp.float32),
            pltpu.VMEM((1, K_EMB), jnp.float32),
            pltpu.SMEM((1,), jnp.float32),
            pltpu.SemaphoreType.DMA((2,)),
            pltpu.SemaphoreType.DMA((2,)),
        ],
    )(x, w1t, b1r, w2t, b2r, codebook, cb2, cs)
    return (loss[0], out[:, :, :T2], perp[0])
